# Initial kernel scaffold; baseline (speedup 1.0000x reference)
import jax
import jax.numpy as jnp
from jax.experimental import pallas as pl


def kernel(dist_edges, dist_vec, poi_table, x, batch, poi, W0, b0, W1, b1, Wqkv, bqkv, Wo, bo):
    raise NotImplementedError("write your pallas kernel here")


# speedup vs baseline: 3.4687x; 3.4687x over previous
import functools

import jax
import jax.numpy as jnp
from jax import lax
from jax.experimental import pallas as pl
from jax.experimental.pallas import tpu as pltpu

N = 10000
NPAD = 10240
D = 128
HEADS = 8
DH = 16
NE = 320000
E = 2 * NE + N
TOK = 25600
B = 512

_f32 = jnp.float32
_i32 = jnp.int32


def _tc_rsq(parts):
    def body(x_ref, o_ref):
        s = x_ref[0] + x_ref[1]
        o_ref[...] = lax.rsqrt(jnp.maximum(s, 1.0))

    return pl.pallas_call(
        body,
        grid=(NPAD // 2048,),
        in_specs=[pl.BlockSpec((2, 2048, 16), lambda i: (0, i, 0))],
        out_specs=pl.BlockSpec((2048, 16), lambda i: (i, 0)),
        out_shape=jax.ShapeDtypeStruct((NPAD, 16), _f32),
    )(parts)


def _tc_linear(x, wt, b, block_rows):
    R = x.shape[0]
    OD = wt.shape[1]

    def body(x_ref, w_ref, b_ref, o_ref):
        o_ref[...] = jnp.dot(x_ref[...], w_ref[...],
                             preferred_element_type=_f32) + b_ref[...]

    return pl.pallas_call(
        body,
        grid=(R // block_rows,),
        in_specs=[
            pl.BlockSpec((block_rows, x.shape[1]), lambda i: (i, 0)),
            pl.BlockSpec((x.shape[1], OD), lambda i: (0, 0)),
            pl.BlockSpec((1, OD), lambda i: (0, 0)),
        ],
        out_specs=pl.BlockSpec((block_rows, OD), lambda i: (i, 0)),
        out_shape=jax.ShapeDtypeStruct((R, OD), _f32),
    )(x, wt, b)


def _tc_linear_scaled(x, wt, b, rsq):
    BR = 2048

    def body(x_ref, w_ref, b_ref, r_ref, o_ref):
        h = jnp.dot(x_ref[...], w_ref[...],
                    preferred_element_type=_f32) + b_ref[...]
        o_ref[...] = h * r_ref[:, :1]

    return pl.pallas_call(
        body,
        grid=(NPAD // BR,),
        in_specs=[
            pl.BlockSpec((BR, D), lambda i: (i, 0)),
            pl.BlockSpec((D, D), lambda i: (0, 0)),
            pl.BlockSpec((1, D), lambda i: (0, 0)),
            pl.BlockSpec((BR, 16), lambda i: (i, 0)),
        ],
        out_specs=pl.BlockSpec((BR, D), lambda i: (i, 0)),
        out_shape=jax.ShapeDtypeStruct((NPAD, D), _f32),
    )(x, wt, b, rsq)


def _tc_post(parts, rsq):
    def body(p_ref, r_ref, o_ref):
        sm = jnp.concatenate([p_ref[0], p_ref[1]], axis=-1) * r_ref[:, :1]
        y = jnp.where(sm >= 0, sm, 0.01 * sm)
        n = jnp.sqrt(jnp.sum(y * y, axis=1, keepdims=True))
        o_ref[...] = y / jnp.maximum(n, 1e-12)

    return pl.pallas_call(
        body,
        grid=(NPAD // 2048,),
        in_specs=[pl.BlockSpec((2, 2048, 64), lambda i: (0, i, 0)),
                  pl.BlockSpec((2048, 16), lambda i: (i, 0))],
        out_specs=pl.BlockSpec((2048, D), lambda i: (i, 0)),
        out_shape=jax.ShapeDtypeStruct((NPAD, D), _f32),
    )(parts, rsq)


def _flash(q, k, v, bq_col, bk_row, m_row, b3, sp, CQ, CK):
    NQ = q.shape[0] // CQ
    NK = k.shape[0] // CK
    scale = 0.25

    def kmap(i, j, sp_ref):
        return (jnp.maximum(jnp.minimum(sp_ref[0, i] + j, sp_ref[1, i]), 0), 0)

    def kmap3(i, j, sp_ref):
        return (jnp.maximum(jnp.minimum(sp_ref[0, i] + j, sp_ref[1, i]), 0), 0, 0)

    def body(sp_ref, q_ref, k_ref, v_ref, bq_ref, bk_ref, m_ref, b3_ref,
             o_ref, m_scr, l_scr, a_scr):
        i = pl.program_id(0)
        j = pl.program_id(1)
        span = sp_ref[1, i] - sp_ref[0, i]
        bq = bq_ref[0]

        @pl.when(j == 0)
        def _init():
            onehot = (bq == lax.broadcasted_iota(_i32, (CQ, B), 1)).astype(_f32)
            mc = jnp.sum(onehot * m_ref[...], axis=1, keepdims=True)
            for h in range(HEADS):
                sl = slice(h * DH, (h + 1) * DH)
                kp = b3_ref[1:2, sl]
                vp = b3_ref[2:3, sl]
                spc = lax.dot_general(q_ref[:, sl], kp,
                                      (((1,), (1,)), ((), ())),
                                      preferred_element_type=_f32) * scale
                m_scr[:, h:h + 1] = jnp.where(mc > 0, spc, -1e30)
                l_scr[:, h:h + 1] = mc
                a_scr[:, sl] = mc * vp

        @pl.when(j <= span)
        def _compute():
            bk = bk_ref[0]
            mask = bq == bk
            for h in range(HEADS):
                sl = slice(h * DH, (h + 1) * DH)
                S = lax.dot_general(q_ref[:, sl], k_ref[:, sl],
                                    (((1,), (1,)), ((), ())),
                                    preferred_element_type=_f32) * scale
                Sm = jnp.where(mask, S, -1e30)
                mold = m_scr[:, h:h + 1]
                mnew = jnp.maximum(mold, jnp.max(Sm, axis=1, keepdims=True))
                p = jnp.where(mask, jnp.exp(Sm - mnew), 0.0)
                corr = jnp.exp(mold - mnew)
                m_scr[:, h:h + 1] = mnew
                l_scr[:, h:h + 1] = (l_scr[:, h:h + 1] * corr
                                     + jnp.sum(p, axis=1, keepdims=True))
                a_scr[:, sl] = (a_scr[:, sl] * corr
                                + lax.dot_general(p, v_ref[:, sl],
                                                  (((1,), (0,)), ((), ())),
                                                  preferred_element_type=_f32))

        @pl.when(j == NK - 1)
        def _fin():
            for h in range(HEADS):
                sl = slice(h * DH, (h + 1) * DH)
                o_ref[:, sl] = a_scr[:, sl] / l_scr[:, h:h + 1]

    gs = pltpu.PrefetchScalarGridSpec(
        num_scalar_prefetch=1,
        grid=(NQ, NK),
        in_specs=[
            pl.BlockSpec((CQ, D), lambda i, j, sp_ref: (i, 0)),
            pl.BlockSpec((CK, D), kmap),
            pl.BlockSpec((CK, D), kmap),
            pl.BlockSpec((1, CQ, 1), lambda i, j, sp_ref: (i, 0, 0)),
            pl.BlockSpec((1, 1, CK), kmap3),
            pl.BlockSpec((1, B), lambda i, j, sp_ref: (0, 0)),
            pl.BlockSpec((3, D), lambda i, j, sp_ref: (0, 0)),
        ],
        out_specs=pl.BlockSpec((CQ, D), lambda i, j, sp_ref: (i, 0)),
        scratch_shapes=[
            pltpu.VMEM((CQ, HEADS), _f32),
            pltpu.VMEM((CQ, HEADS), _f32),
            pltpu.VMEM((CQ, D), _f32),
        ],
    )
    return pl.pallas_call(
        body, grid_spec=gs,
        out_shape=jax.ShapeDtypeStruct((NQ * CQ, D), _f32),
    )(sp, q, k, v, bq_col, bk_row, m_row, b3)


def _tc_aggr(o_tok, bt_rows, o_pad, m_col, wot, bo, ml):
    CT = 1600
    NT = TOK // CT

    def body(ot_ref, bt_ref, op_ref, m_ref, w_ref, b_ref, ml_ref, o_ref, acc):
        t = pl.program_id(0)

        @pl.when(t == 0)
        def _z():
            acc[...] = jnp.zeros((B, D), _f32)

        bt = bt_ref[0]
        oh = (lax.broadcasted_iota(_i32, (B, CT), 0) == bt).astype(_f32)
        acc[...] += jnp.dot(oh, ot_ref[...], preferred_element_type=_f32)

        @pl.when(t == NT - 1)
        def _fin():
            A = (acc[...] + m_ref[...] * op_ref[...]) / ml_ref[0, 0]
            o_ref[...] = jnp.dot(A, w_ref[...],
                                 preferred_element_type=_f32) + b_ref[...]

    return pl.pallas_call(
        body,
        grid=(NT,),
        in_specs=[
            pl.BlockSpec((CT, D), lambda t: (t, 0)),
            pl.BlockSpec((1, 1, CT), lambda t: (t, 0, 0)),
            pl.BlockSpec((B, D), lambda t: (0, 0)),
            pl.BlockSpec((B, 1), lambda t: (0, 0)),
            pl.BlockSpec((D, D), lambda t: (0, 0)),
            pl.BlockSpec((1, D), lambda t: (0, 0)),
            pl.BlockSpec((1, 1), lambda t: (0, 0)),
        ],
        out_specs=pl.BlockSpec((B, D), lambda t: (0, 0)),
        out_shape=jax.ShapeDtypeStruct((B, D), _f32),
        scratch_shapes=[pltpu.VMEM((B, D), _f32)],
    )(o_tok, bt_rows, o_pad, m_col, wot, bo, ml)


def kernel(dist_edges, dist_vec, poi_table, x, batch, poi,
           W0, b0, W1, b1, Wqkv, bqkv, Wo, bo):
    loop = jnp.arange(N, dtype=dist_edges.dtype)
    src = jnp.concatenate([dist_edges[0], dist_edges[1], loop])
    dst = jnp.concatenate([dist_edges[1], dist_edges[0], loop])
    dvf = jnp.concatenate([dist_vec, dist_vec, jnp.zeros((N,), _f32)])

    deg = jnp.zeros((NPAD,), _f32).at[src].add(1.0)
    deg_parts = jnp.stack([jnp.broadcast_to(deg[:, None], (NPAD, 16)),
                           jnp.zeros((NPAD, 16), _f32)])
    rsq = _tc_rsq(deg_parts)
    w = jnp.exp(-dvf * dvf)

    enc = jnp.concatenate([poi_table, jnp.zeros((NPAD - N, D), _f32)], axis=0)
    b0r = b0.reshape(1, D)
    b1r = b1.reshape(1, D)

    for (W, br) in ((W0, b0r), (W1, b1r)):
        h = _tc_linear_scaled(enc, W.T, br, rsq)
        out = jnp.zeros((NPAD, D), _f32).at[src].add(w[:, None] * h[dst])
        parts = jnp.stack([out[:, :64], out[:, 64:]])
        enc = _tc_post(parts, rsq)

    seq = enc[x]
    tar = enc[poi]

    qkv = _tc_linear(seq, Wqkv.T, bqkv.reshape(1, 3 * D), 3200)
    q, k, v = qkv[:, :D], qkv[:, D:2 * D], qkv[:, 2 * D:]

    offsets = jnp.searchsorted(batch, jnp.arange(B + 1, dtype=batch.dtype)
                               ).astype(_i32)
    counts = offsets[1:] - offsets[:-1]
    max_len = jnp.max(counts)
    m = (max_len - counts).astype(_f32)

    CQ, CK = 1024, 1024
    NQ, NK = TOK // CQ, TOK // CK
    lo_tok = offsets[batch[::CQ]]
    hi_tok = offsets[batch[CQ - 1::CQ] + 1] - 1
    spA = jnp.stack([lo_tok // CK, hi_tok // CK]).astype(_i32)

    batch32 = batch.astype(_i32)
    bq_col = batch32.reshape(NQ, CQ, 1)
    bk_row = batch32.reshape(NK, 1, CK)
    m_row = m.reshape(1, B)
    b3 = bqkv.reshape(3, D)

    o_tok = _flash(q, k, v, bq_col, bk_row, m_row, b3, spA, CQ, CK)

    BB = 64
    NB = B // BB
    loB = offsets[:B:BB]
    hiB = offsets[BB::BB] - 1
    spB = jnp.stack([loB // CK, hiB // CK]).astype(_i32)
    q_pad = jnp.broadcast_to(bqkv[:D].reshape(1, D), (B, D))
    bqB = jnp.arange(B, dtype=_i32).reshape(NB, BB, 1)
    o_pad = _flash(q_pad, k, v, bqB, bk_row, m_row, b3, spB, BB, CK)

    aggr = _tc_aggr(o_tok, batch32.reshape(16, 1, 1600), o_pad,
                    m.reshape(B, 1), Wo.T, bo.reshape(1, D),
                    max_len.astype(_f32).reshape(1, 1))
    return (aggr, tar)
